# baseline (speedup 1.0000x reference)
import jax
import jax.numpy as jnp
from jax import lax
from jax.experimental import pallas as pl
from jax.experimental.pallas import tpu as pltpu
from jax.experimental.pallas import tpu_sc as plsc

VOCAB = 1000000
DIM = 64
BATCH = 16384
NEG = 20

NC = 2
NS = 16
LANES = 16
NPART = DIM // LANES
NW = NC * NS
B_PER_W = BATCH // NW
CB = 32
NCHUNK = B_PER_W // CB
NEG_STREAMS = CB * NEG // 128
PACK = 32
FILL = PACK - (NEG + 1)
PROW = 2 * DIM


PBLK = 512
PGRID = (VOCAB + 2 * PBLK - 1) // (2 * PBLK)
PV = PGRID * PBLK


def _pair_body(xa_ref, xb_ref, o_ref):
    o_ref[...] = jnp.concatenate([xa_ref[...], xb_ref[...]], axis=1)


def _to_pairs(table):
    return pl.pallas_call(
        _pair_body,
        grid=(PGRID,),
        in_specs=[pl.BlockSpec((PBLK, DIM), lambda i: (2 * i, 0)),
                  pl.BlockSpec((PBLK, DIM), lambda i: (2 * i + 1, 0))],
        out_specs=pl.BlockSpec((PBLK, PROW), lambda i: (i, 0)),
        out_shape=jax.ShapeDtypeStruct((PV, PROW), jnp.float32),
    )(table, table)


def _hsum(acc):
    for sh in (8, 4, 2, 1):
        perm = lax.iota(jnp.int32, LANES) ^ sh
        acc = acc + acc.at[perm].get(mode="promise_in_bounds")
    return acc


def _pair_of(v):
    return ((v >> 10) << 9) | (v & 511)


def _half_off(v):
    return ((v >> 9) & 1) * DIM


def _sc_body(tgt_hbm, ctx_hbm, neg_hbm, in_emb, out_emb, scores_out,
             tidx, cidx, nidx, ptidx, pcidx, pnidx,
             t_rows, c_rows, n_rows, score_buf, sem):
    wid = lax.axis_index("s") * NC + lax.axis_index("c")

    def chunk_body(ci, _):
        gbase = wid * B_PER_W + ci * CB

        pltpu.sync_copy(tgt_hbm.at[pl.ds(gbase, CB)], tidx.at[pl.ds(0, CB)])
        pltpu.sync_copy(ctx_hbm.at[pl.ds(gbase, CB)], cidx.at[pl.ds(0, CB)])
        for j in range(NEG_STREAMS):
            pltpu.sync_copy(neg_hbm.at[pl.ds(gbase * NEG + j * 128, 128)],
                            nidx.at[pl.ds(j * 128, 128)])

        for l in range(CB // LANES):
            v = tidx[pl.ds(l * LANES, LANES)]
            ptidx[pl.ds(l * LANES, LANES)] = _pair_of(v)
            v = cidx[pl.ds(l * LANES, LANES)]
            pcidx[pl.ds(l * LANES, LANES)] = _pair_of(v)
        for j in range(NEG_STREAMS):
            for l in range(128 // LANES):
                v = nidx[pl.ds(j * 128 + l * LANES, LANES)]
                pnidx[j, pl.ds(l * LANES, LANES)] = _pair_of(v)

        copies = [pltpu.async_copy(in_emb.at[ptidx], t_rows, sem),
                  pltpu.async_copy(out_emb.at[pcidx], c_rows, sem)]
        for j in range(NEG_STREAMS):
            copies.append(pltpu.async_copy(
                out_emb.at[pnidx.at[j]], n_rows.at[pl.ds(j * 128, 128)], sem))
        for c in copies:
            c.wait()

        def elem_body(b, _):
            toff = _half_off(tidx[pl.ds(b, LANES)][0])
            coff = _half_off(cidx[pl.ds(b, LANES)][0])
            t = [t_rows[b, pl.ds(toff + i * LANES, LANES)]
                 for i in range(NPART)]
            tn = [-x for x in t]
            c = [c_rows[b, pl.ds(coff + i * LANES, LANES)]
                 for i in range(NPART)]

            def dot(a_parts, b_parts):
                acc = a_parts[0] * b_parts[0]
                for i in range(1, NPART):
                    acc = acc + a_parts[i] * b_parts[i]
                return _hsum(acc)

            def neg_dot(k):
                r = b * NEG + k
                noff = _half_off(nidx[pl.ds(r, LANES)][0])
                n = [n_rows[r, pl.ds(noff + i * LANES, LANES)]
                     for i in range(NPART)]
                return dot(tn, n)

            pack_a = dot(t, c)
            for k in range(15):
                mask = lax.iota(jnp.int32, LANES) == (k + 1)
                pack_a = jnp.where(mask, neg_dot(k), pack_a)
            pack_b = jnp.zeros((LANES,), jnp.float32)
            for k in range(15, NEG):
                mask = lax.iota(jnp.int32, LANES) == (k - 15)
                pack_b = jnp.where(mask, neg_dot(k), pack_b)

            score_buf[pl.ds(b * PACK, LANES)] = pack_a
            score_buf[pl.ds(b * PACK + LANES, LANES)] = pack_b
            return ()

        lax.fori_loop(0, CB, elem_body, (), unroll=False)

        pltpu.sync_copy(score_buf, scores_out.at[pl.ds(gbase * PACK, CB * PACK)])
        return ()

    lax.fori_loop(0, NCHUNK, chunk_body, (), unroll=False)


def _scores_sc(tgt, ctx, negs, in_emb, out_emb):
    mesh = plsc.VectorSubcoreMesh(core_axis_name="c", subcore_axis_name="s")
    f = pl.kernel(
        _sc_body,
        out_type=jax.ShapeDtypeStruct((BATCH * PACK,), jnp.float32),
        mesh=mesh,
        scratch_types=[
            pltpu.VMEM((CB + LANES,), jnp.int32),
            pltpu.VMEM((CB + LANES,), jnp.int32),
            pltpu.VMEM((NEG_STREAMS * 128 + LANES,), jnp.int32),
            pltpu.VMEM((CB,), jnp.int32),
            pltpu.VMEM((CB,), jnp.int32),
            pltpu.VMEM((NEG_STREAMS, 128), jnp.int32),
            pltpu.VMEM((CB, PROW), jnp.float32),
            pltpu.VMEM((CB, PROW), jnp.float32),
            pltpu.VMEM((CB * NEG, PROW), jnp.float32),
            pltpu.VMEM((CB * PACK,), jnp.float32),
            pltpu.SemaphoreType.DMA,
        ],
        compiler_params=pltpu.CompilerParams(use_tc_tiling_on_sc=True),
    )
    return f(tgt, ctx, negs, in_emb, out_emb)


def _loss_body(y_ref, out_ref):
    total = jnp.sum(jax.nn.log_sigmoid(y_ref[...]))
    valid = total + FILL * BATCH * jnp.float32(jnp.log(2.0))
    out_ref[0, 0] = -valid / BATCH


def _loss_tc(scores):
    out = pl.pallas_call(
        _loss_body,
        out_shape=jax.ShapeDtypeStruct((1, 1), jnp.float32),
        in_specs=[pl.BlockSpec(memory_space=pltpu.VMEM)],
        out_specs=pl.BlockSpec(memory_space=pltpu.SMEM),
    )(scores.reshape(BATCH * PACK // 128, 128))
    return out[0, 0]


@jax.jit
def kernel(target_word, context_word, negative_words,
           input_embeddings, output_embeddings):
    tgt = target_word.astype(jnp.int32)
    ctx = context_word.astype(jnp.int32)
    negs = negative_words.astype(jnp.int32).reshape(BATCH * NEG)
    in_p = _to_pairs(input_embeddings)
    out_p = _to_pairs(output_embeddings)
    scores = _scores_sc(tgt, ctx, negs, in_p, out_p)
    return _loss_tc(scores)


# speedup vs baseline: 1.9904x; 1.9904x over previous
import jax
import jax.numpy as jnp
from jax import lax
from jax.experimental import pallas as pl
from jax.experimental.pallas import tpu as pltpu
from jax.experimental.pallas import tpu_sc as plsc

VOCAB = 1000000
DIM = 64
BATCH = 16384
NEG = 20

NC = 2
NS = 16
LANES = 16
NPART = DIM // LANES
NW = NC * NS
B_PER_W = BATCH // NW
CB = 16
NCHUNK = B_PER_W // CB
NSTREAM = 4
NSIDX = CB * NEG // NSTREAM
PACK = 32
FILL = PACK - (NEG + 1)
PROW = 2 * DIM


def _hsum(acc):
    for sh in (8, 4, 2, 1):
        perm = lax.iota(jnp.int32, LANES) ^ sh
        acc = acc + acc.at[perm].get(mode="promise_in_bounds")
    return acc


def _sc_body(tgt_hbm, ctx_hbm, neg_hbm, in_emb, out_emb, scores_out,
             tidx, cidx, nidx, t_rows, c_rows, n_rows, score_buf,
             sg0, sg1, sw0, sw1):
    wid = lax.axis_index("s") * NC + lax.axis_index("c")
    base = wid * B_PER_W
    sg = (sg0, sg1)
    sw = (sw0, sw1)

    for cp in [pltpu.async_copy(tgt_hbm.at[pl.ds(base, B_PER_W)], tidx, sg0),
               pltpu.async_copy(ctx_hbm.at[pl.ds(base, B_PER_W)], cidx, sg0),
               pltpu.async_copy(neg_hbm.at[pl.ds(base * NEG, B_PER_W * NEG)],
                                nidx, sg0)]:
        cp.wait()

    def gather_args(c, slot):
        args = [(in_emb.at[tidx.at[pl.ds(c * CB, CB)]], t_rows.at[slot]),
                (out_emb.at[cidx.at[pl.ds(c * CB, CB)]], c_rows.at[slot])]
        for j in range(NSTREAM):
            args.append(
                (out_emb.at[nidx.at[pl.ds(c * CB * NEG + j * NSIDX, NSIDX)]],
                 n_rows.at[slot].at[pl.ds(j * NSIDX, NSIDX)]))
        return args

    def fire_gathers(c, slot):
        for src, dst in gather_args(c, slot):
            pltpu.async_copy(src, dst, sg[slot])

    def wait_gathers(c, slot):
        for src, dst in gather_args(c, slot):
            pltpu.make_async_copy(src, dst, sg[slot]).wait()

    def out_slice(c):
        return scores_out.at[pl.ds((base + c * CB) * PACK, CB * PACK)]

    def compute(c, slot):
        def elem_body(b, _):
            t = [t_rows[slot, b, pl.ds(i * LANES, LANES)]
                 for i in range(NPART)]
            tn = [-x for x in t]
            cv = [c_rows[slot, b, pl.ds(i * LANES, LANES)]
                  for i in range(NPART)]

            def dot(a_parts, b_parts):
                acc = a_parts[0] * b_parts[0]
                for i in range(1, NPART):
                    acc = acc + a_parts[i] * b_parts[i]
                return _hsum(acc)

            def neg_dot(k):
                n = [n_rows[slot, b * NEG + k, pl.ds(i * LANES, LANES)]
                     for i in range(NPART)]
                return dot(tn, n)

            pack_a = dot(t, cv)
            for k in range(15):
                mask = lax.iota(jnp.int32, LANES) == (k + 1)
                pack_a = jnp.where(mask, neg_dot(k), pack_a)
            pack_b = jnp.zeros((LANES,), jnp.float32)
            for k in range(15, NEG):
                mask = lax.iota(jnp.int32, LANES) == (k - 15)
                pack_b = jnp.where(mask, neg_dot(k), pack_b)

            score_buf[slot, pl.ds(b * PACK, LANES)] = pack_a
            score_buf[slot, pl.ds(b * PACK + LANES, LANES)] = pack_b
            return ()

        lax.fori_loop(0, CB, elem_body, (), unroll=False)

    def process(c, slot, other):
        wait_gathers(c, slot)

        @pl.when(c + 1 < NCHUNK)
        def _():
            fire_gathers(c + 1, other)

        @pl.when(c >= 2)
        def _():
            pltpu.make_async_copy(score_buf.at[slot], out_slice(c - 2),
                                  sw[slot]).wait()

        compute(c, slot)
        pltpu.async_copy(score_buf.at[slot], out_slice(c), sw[slot])

    fire_gathers(0, 0)

    def pair_body(tt, _):
        process(2 * tt, 0, 1)
        process(2 * tt + 1, 1, 0)
        return ()

    lax.fori_loop(0, NCHUNK // 2, pair_body, (), unroll=False)

    pltpu.make_async_copy(score_buf.at[0], out_slice(NCHUNK - 2), sw0).wait()
    pltpu.make_async_copy(score_buf.at[1], out_slice(NCHUNK - 1), sw1).wait()


def _scores_sc(tgt, ctx, negs, in_emb, out_emb):
    mesh = plsc.VectorSubcoreMesh(core_axis_name="c", subcore_axis_name="s")
    f = pl.kernel(
        _sc_body,
        out_type=jax.ShapeDtypeStruct((BATCH * PACK,), jnp.float32),
        mesh=mesh,
        scratch_types=[
            pltpu.VMEM((B_PER_W,), jnp.int32),
            pltpu.VMEM((B_PER_W,), jnp.int32),
            pltpu.VMEM((B_PER_W * NEG,), jnp.int32),
            pltpu.VMEM((2, CB, PROW), jnp.float32),
            pltpu.VMEM((2, CB, PROW), jnp.float32),
            pltpu.VMEM((2, CB * NEG, PROW), jnp.float32),
            pltpu.VMEM((2, CB * PACK), jnp.float32),
            pltpu.SemaphoreType.DMA,
            pltpu.SemaphoreType.DMA,
            pltpu.SemaphoreType.DMA,
            pltpu.SemaphoreType.DMA,
        ],
        compiler_params=pltpu.CompilerParams(use_tc_tiling_on_sc=True),
    )
    return f(tgt, ctx, negs, in_emb, out_emb)


def _loss_body(y_ref, out_ref):
    total = jnp.sum(jax.nn.log_sigmoid(y_ref[...]))
    valid = total + FILL * BATCH * jnp.float32(jnp.log(2.0))
    out_ref[0, 0] = -valid / BATCH


def _loss_tc(scores):
    out = pl.pallas_call(
        _loss_body,
        out_shape=jax.ShapeDtypeStruct((1, 1), jnp.float32),
        in_specs=[pl.BlockSpec(memory_space=pltpu.VMEM)],
        out_specs=pl.BlockSpec(memory_space=pltpu.SMEM),
    )(scores.reshape(BATCH * PACK // 128, 128))
    return out[0, 0]


@jax.jit
def kernel(target_word, context_word, negative_words,
           input_embeddings, output_embeddings):
    tgt = target_word.astype(jnp.int32)
    ctx = context_word.astype(jnp.int32)
    negs = negative_words.astype(jnp.int32).reshape(BATCH * NEG)
    in_p = jnp.pad(input_embeddings, ((0, 0), (0, PROW - DIM)))
    out_p = jnp.pad(output_embeddings, ((0, 0), (0, PROW - DIM)))
    scores = _scores_sc(tgt, ctx, negs, in_p, out_p)
    return _loss_tc(scores)


# speedup vs baseline: 2.3736x; 1.1925x over previous
import jax
import jax.numpy as jnp
from jax import lax
from jax.experimental import pallas as pl
from jax.experimental.pallas import tpu as pltpu
from jax.experimental.pallas import tpu_sc as plsc

VOCAB = 1000000
DIM = 64
BATCH = 16384
NEG = 20

NC = 2
NS = 16
LANES = 16
NPART = DIM // LANES
NW = NC * NS
B_PER_W = BATCH // NW
CB = 8
NCHUNK = B_PER_W // CB
NSTREAM = 4
NSIDX = CB * NEG // NSTREAM
PACK = 32
FILL = PACK - (NEG + 1)
PROW = 2 * DIM


def _hsum(acc):
    for sh in (8, 4, 2, 1):
        perm = lax.iota(jnp.int32, LANES) ^ sh
        acc = acc + acc.at[perm].get(mode="promise_in_bounds")
    return acc


def _sc_body(tgt_hbm, ctx_hbm, neg_hbm, in_emb, out_emb, scores_out,
             tidx, cidx, nidx, t_rows, c_rows, n_rows, score_buf,
             sg0, sg1, sw0, sw1):
    wid = lax.axis_index("s") * NC + lax.axis_index("c")
    base = wid * B_PER_W
    sg = (sg0, sg1)
    sw = (sw0, sw1)

    for cp in [pltpu.async_copy(tgt_hbm.at[pl.ds(base, B_PER_W)],
                                tidx.at[pl.ds(0, B_PER_W)], sg0),
               pltpu.async_copy(ctx_hbm.at[pl.ds(base, B_PER_W)], cidx, sg0),
               pltpu.async_copy(neg_hbm.at[pl.ds(base * NEG, B_PER_W * NEG)],
                                nidx, sg0)]:
        cp.wait()

    def gather_args(c, slot):
        args = [(out_emb.at[cidx.at[pl.ds(c * CB, CB)]], c_rows.at[slot])]
        for j in range(NSTREAM):
            args.append(
                (out_emb.at[nidx.at[pl.ds(c * CB * NEG + j * NSIDX, NSIDX)]],
                 n_rows.at[slot].at[pl.ds(j * NSIDX, NSIDX)]))
        return args

    def t_window(c, b):
        v = tidx[pl.ds(c * CB + b, LANES)][0]
        return pl.multiple_of(v & -8, 8)

    def fire_gathers(c, slot):
        for b in range(CB):
            pltpu.async_copy(in_emb.at[pl.ds(t_window(c, b), 8)],
                             t_rows.at[slot].at[pl.ds(b * 8, 8)], sg[slot])
        for src, dst in gather_args(c, slot):
            pltpu.async_copy(src, dst, sg[slot])

    def wait_gathers(c, slot):
        for b in range(CB):
            pltpu.make_async_copy(in_emb.at[pl.ds(0, 8)],
                                  t_rows.at[slot].at[pl.ds(b * 8, 8)],
                                  sg[slot]).wait()
        for src, dst in gather_args(c, slot):
            pltpu.make_async_copy(src, dst, sg[slot]).wait()

    def out_slice(c):
        return scores_out.at[pl.ds((base + c * CB) * PACK, CB * PACK)]

    def compute(c, slot):
        def elem_body(b, _):
            tv = tidx[pl.ds(c * CB + b, LANES)][0]
            trow = b * 8 + (tv & 7)
            t = [t_rows[slot, trow, pl.ds(i * LANES, LANES)]
                 for i in range(NPART)]
            tn = [-x for x in t]
            cv = [c_rows[slot, b, pl.ds(i * LANES, LANES)]
                  for i in range(NPART)]

            def dot(a_parts, b_parts):
                acc = a_parts[0] * b_parts[0]
                for i in range(1, NPART):
                    acc = acc + a_parts[i] * b_parts[i]
                return _hsum(acc)

            def neg_dot(k):
                n = [n_rows[slot, b * NEG + k, pl.ds(i * LANES, LANES)]
                     for i in range(NPART)]
                return dot(tn, n)

            pack_a = dot(t, cv)
            for k in range(15):
                mask = lax.iota(jnp.int32, LANES) == (k + 1)
                pack_a = jnp.where(mask, neg_dot(k), pack_a)
            pack_b = jnp.zeros((LANES,), jnp.float32)
            for k in range(15, NEG):
                mask = lax.iota(jnp.int32, LANES) == (k - 15)
                pack_b = jnp.where(mask, neg_dot(k), pack_b)

            score_buf[slot, pl.ds(b * PACK, LANES)] = pack_a
            score_buf[slot, pl.ds(b * PACK + LANES, LANES)] = pack_b
            return ()

        lax.fori_loop(0, CB, elem_body, (), unroll=False)

    def process(c, slot, other):
        wait_gathers(c, slot)

        @pl.when(c + 1 < NCHUNK)
        def _():
            fire_gathers(c + 1, other)

        @pl.when(c >= 2)
        def _():
            pltpu.make_async_copy(score_buf.at[slot], out_slice(c - 2),
                                  sw[slot]).wait()

        compute(c, slot)
        pltpu.async_copy(score_buf.at[slot], out_slice(c), sw[slot])

    fire_gathers(0, 0)

    def pair_body(tt, _):
        process(2 * tt, 0, 1)
        process(2 * tt + 1, 1, 0)
        return ()

    lax.fori_loop(0, NCHUNK // 2, pair_body, (), unroll=False)

    pltpu.make_async_copy(score_buf.at[0], out_slice(NCHUNK - 2), sw0).wait()
    pltpu.make_async_copy(score_buf.at[1], out_slice(NCHUNK - 1), sw1).wait()


def _scores_sc(tgt, ctx, negs, in_emb, out_emb):
    mesh = plsc.VectorSubcoreMesh(core_axis_name="c", subcore_axis_name="s")
    f = pl.kernel(
        _sc_body,
        out_type=jax.ShapeDtypeStruct((BATCH * PACK,), jnp.float32),
        mesh=mesh,
        scratch_types=[
            pltpu.VMEM((B_PER_W + LANES,), jnp.int32),
            pltpu.VMEM((B_PER_W,), jnp.int32),
            pltpu.VMEM((B_PER_W * NEG,), jnp.int32),
            pltpu.VMEM((2, CB * 8, DIM), jnp.float32),
            pltpu.VMEM((2, CB, PROW), jnp.float32),
            pltpu.VMEM((2, CB * NEG, PROW), jnp.float32),
            pltpu.VMEM((2, CB * PACK), jnp.float32),
            pltpu.SemaphoreType.DMA,
            pltpu.SemaphoreType.DMA,
            pltpu.SemaphoreType.DMA,
            pltpu.SemaphoreType.DMA,
        ],
        compiler_params=pltpu.CompilerParams(use_tc_tiling_on_sc=True),
    )
    return f(tgt, ctx, negs, in_emb, out_emb)


def _loss_body(y_ref, out_ref):
    total = jnp.sum(jax.nn.log_sigmoid(y_ref[...]))
    valid = total + FILL * BATCH * jnp.float32(jnp.log(2.0))
    out_ref[0, 0] = -valid / BATCH


def _loss_tc(scores):
    out = pl.pallas_call(
        _loss_body,
        out_shape=jax.ShapeDtypeStruct((1, 1), jnp.float32),
        in_specs=[pl.BlockSpec(memory_space=pltpu.VMEM)],
        out_specs=pl.BlockSpec(memory_space=pltpu.SMEM),
    )(scores.reshape(BATCH * PACK // 128, 128))
    return out[0, 0]


@jax.jit
def kernel(target_word, context_word, negative_words,
           input_embeddings, output_embeddings):
    tgt = target_word.astype(jnp.int32)
    ctx = context_word.astype(jnp.int32)
    negs = negative_words.astype(jnp.int32).reshape(BATCH * NEG)
    out_p = jnp.pad(output_embeddings, ((0, 0), (0, PROW - DIM)))
    scores = _scores_sc(tgt, ctx, negs, input_embeddings, out_p)
    return _loss_tc(scores)
